# baseline (speedup 1.0000x reference)
import functools
import jax
import jax.numpy as jnp
from jax import lax
from jax.experimental import pallas as pl
from jax.experimental.pallas import tpu as pltpu
from jax.experimental.pallas import tpu_sc as plsc

NC, NS, L = 2, 16, 16
NW = NC * NS
PW_ = 1.0

f32 = jnp.float32
i32 = jnp.int32


def _build_sc_call(T, P, BINS, NN, EV_CHUNK=1024):
    PB = P // NW
    NBK = BINS * PB
    PBF = 2 * PB
    NG = PBF // L
    NTILE = (NN + 127) // 128
    bwf = f32(1.0 / BINS)
    half = f32(0.5 / BINS)
    mesh = plsc.VectorSubcoreMesh(core_axis_name="c", subcore_axis_name="s")

    @functools.partial(
        pl.kernel, mesh=mesh,
        out_type=jax.ShapeDtypeStruct((NW * L,), f32),
        compiler_params=pltpu.CompilerParams(needs_layout_passes=False),
        scratch_types=[
            pltpu.VMEM((136,), i32),
            pltpu.VMEM((PB,), i32),
            pltpu.VMEM((PB,), i32),
            pltpu.VMEM((PB,), i32),
            pltpu.VMEM((PB,), i32),
            pltpu.VMEM((PB,), i32),
            pltpu.VMEM((PB,), i32),
            pltpu.VMEM((PB,), i32),
            pltpu.VMEM((PB,), i32),
            pltpu.VMEM((PB,), i32),
            pltpu.VMEM((PB,), i32),
            pltpu.VMEM((PBF,), i32),
            pltpu.VMEM((PBF,), i32),
            pltpu.VMEM((PB, 128), f32),
            pltpu.VMEM((PB, 128), f32),
            pltpu.VMEM((PB, 128), f32),
            pltpu.VMEM((PB, 128), f32),
            pltpu.VMEM((PBF,), f32),
            pltpu.VMEM((NBK,), f32),
            pltpu.VMEM((NBK,), f32),
            pltpu.VMEM((NBK,), f32),
            pltpu.VMEM((EV_CHUNK,), f32),
            pltpu.VMEM((L,), f32),
            pltpu.VMEM((L,), f32),
            pltpu.SemaphoreType.DMA,
            pltpu.SemaphoreType.DMA,
        ],
    )
    def sc_call(et_hbm, cs_hbm, np_hbm, x0r_hbm, vr_hbm, out_hbm,
                cs_loc, mi_v, mj_v, rtiA_v, rtiB_v, rtjA_v, rtjB_v,
                ix0i_v, ix1i_v, ix0j_v, ix1j_v,
                fci_v, fcj_v, g0i_v, g1i_v, g0j_v, g1j_v, dx0_v,
                cnt_v, s1_v, s2_v, ev_v, shuf_v, orow_v, semA, semB):
        wid = lax.axis_index("s") * NC + lax.axis_index("c")
        pbase = wid * PB
        LANE = lax.iota(i32, L)
        DUPV = lax.shift_right_logical(LANE, 1)
        SWAPV = lax.bitwise_xor(LANE, 1)
        COLV = lax.bitwise_and(LANE, 1)

        pltpu.sync_copy(cs_hbm.at[pl.ds(pbase, 136)], cs_loc)
        pltpu.sync_copy(np_hbm.at[pl.ds(pbase, PB)], mi_v)
        pltpu.sync_copy(np_hbm.at[pl.ds(P + pbase, PB)], mj_v)

        for g in range(NG):
            sl = pl.ds(g * L, L)
            ni = plsc.load_gather(mi_v, [g * 8 + DUPV])
            nj = plsc.load_gather(mj_v, [g * 8 + DUPV])
            fci_v[sl] = lax.bitwise_and(ni, 127)
            fcj_v[sl] = lax.bitwise_and(nj, 127)
            rti = 2 * lax.shift_right_logical(ni, 7) + COLV
            rtj = 2 * lax.shift_right_logical(nj, 7) + COLV
            slh = pl.ds((g % 8) * L, L)
            if g < 8:
                rtiA_v[slh] = rti
                rtjA_v[slh] = rtj
            else:
                rtiB_v[slh] = rti
                rtjB_v[slh] = rtj

        cp0 = pltpu.async_copy(x0r_hbm.at[rtiA_v], g0i_v, semA)
        cp1 = pltpu.async_copy(x0r_hbm.at[rtjA_v], g0j_v, semA)
        cp2 = pltpu.async_copy(x0r_hbm.at[rtiB_v], g1i_v, semB)
        cp3 = pltpu.async_copy(x0r_hbm.at[rtjB_v], g1j_v, semB)
        cp0.wait()
        cp1.wait()
        cp2.wait()
        cp3.wait()

        e0 = cs_loc[pl.ds(0, L)][0]
        e1 = cs_loc[pl.ds(120, L)][8]

        pr0 = jnp.zeros((L,), f32)
        for g in range(NG):
            sl = pl.ds(g * L, L)
            rowv = (g % 8) * L + LANE
            if g < 8:
                xi = plsc.load_gather(g0i_v, [rowv, fci_v[sl]])
                xj = plsc.load_gather(g0j_v, [rowv, fcj_v[sl]])
            else:
                xi = plsc.load_gather(g1i_v, [rowv, fci_v[sl]])
                xj = plsc.load_gather(g1j_v, [rowv, fcj_v[sl]])
            dx0_v[sl] = xi - xj
            pr0 = pr0 + xi * xi + xj * xj

        def zero_body(k, _):
            z = jnp.zeros((L,), f32)
            cnt_v[pl.ds(k * L, L)] = z
            s1_v[pl.ds(k * L, L)] = z
            s2_v[pl.ds(k * L, L)] = z
            return 0
        lax.fori_loop(0, NBK // L, zero_body, 0)

        e0_al = (e0 // 8) * 8
        nch = (e1 - e0_al + (EV_CHUNK - 1)) // EV_CHUNK
        ones = jnp.ones((L,), f32)

        def ev_chunk(ch, _):
            chb = e0_al + ch * EV_CHUNK
            chb_dma = jnp.minimum(chb, T - EV_CHUNK)
            delta = chb - chb_dma
            pltpu.async_copy(
                et_hbm.at[pl.ds(chb_dma, EV_CHUNK)], ev_v, semA).wait()
            for g in range(EV_CHUNK // L):
                gidx = chb + g * L + LANE
                valid = jnp.logical_and(gidx >= e0, gidx < e1)
                t_e = plsc.load_gather(
                    ev_v,
                    [jnp.minimum(delta + g * L + LANE, EV_CHUNK - 1)])
                pos = jnp.zeros((L,), i32)
                for w in (64, 32, 16, 8, 4, 2, 1):
                    cand = pos + w
                    cv = plsc.load_gather(cs_loc, [cand])
                    pos = jnp.where(cv <= gidx, cand, pos)
                b = jnp.clip((t_e / bwf).astype(i32), 0, BINS - 1)
                r = t_e - b.astype(f32) * bwf
                bucket = b * PB + pos
                plsc.addupdate_scatter(cnt_v, [bucket], ones, mask=valid)
                plsc.addupdate_scatter(s1_v, [bucket], r, mask=valid)
                plsc.addupdate_scatter(s2_v, [bucket], r * r, mask=valid)
            return 0
        lax.fori_loop(0, nch, ev_chunk, 0)

        def fire0(b):
            rowoff = 2 * jnp.minimum(b, BINS - 1) * NTILE
            for g in range(PB // L):
                sl = pl.ds(g * L, L)
                ix0i_v[sl] = rtiA_v[sl] + rowoff
                ix0j_v[sl] = rtjA_v[sl] + rowoff
            pltpu.async_copy(vr_hbm.at[ix0i_v], g0i_v, semA)
            pltpu.async_copy(vr_hbm.at[ix0j_v], g0j_v, semA)

        def fire1(b):
            rowoff = 2 * b * NTILE
            for g in range(PB // L):
                sl = pl.ds(g * L, L)
                ix1i_v[sl] = rtiB_v[sl] + rowoff
                ix1j_v[sl] = rtjB_v[sl] + rowoff
            pltpu.async_copy(vr_hbm.at[ix1i_v], g1i_v, semB)
            pltpu.async_copy(vr_hbm.at[ix1j_v], g1j_v, semB)

        def wait0():
            pltpu.make_async_copy(vr_hbm.at[ix0i_v], g0i_v, semA).wait()
            pltpu.make_async_copy(vr_hbm.at[ix0j_v], g0j_v, semA).wait()

        def wait1():
            pltpu.make_async_copy(vr_hbm.at[ix1i_v], g1i_v, semB).wait()
            pltpu.make_async_copy(vr_hbm.at[ix1j_v], g1j_v, semB).wait()

        def extract(b, carry, gsrc_i, gsrc_j, g_lo):
            cums = list(carry[0])
            ev_acc, int_acc, pr_acc = carry[1], carry[2], carry[3]
            for gg in range(NG // 2):
                g = g_lo + gg
                sl = pl.ds(g * L, L)
                rowv = gg * L + LANE
                vi = plsc.load_gather(gsrc_i, [rowv, fci_v[sl]])
                vj = plsc.load_gather(gsrc_j, [rowv, fcj_v[sl]])
                dv = vi - vj
                e_ = dx0_v[sl] + bwf * cums[g]
                cums[g] = cums[g] + dv
                pa = e_ * e_
                pb2 = e_ * dv
                pc = dv * dv
                dup = b * PB + g * 8 + DUPV
                c_ = plsc.load_gather(cnt_v, [dup])
                s1 = plsc.load_gather(s1_v, [dup])
                s2 = plsc.load_gather(s2_v, [dup])
                ev_acc = ev_acc + pa * c_ + 2.0 * pb2 * s1 + pc * s2
                d2l = pa + pb2 * bwf + pc * (half * half)
                shuf_v[...] = d2l
                d2s = plsc.load_gather(shuf_v, [SWAPV])
                int_acc = int_acc + jnp.exp(-(d2l + d2s))
                pr_acc = pr_acc + vi * vi + vj * vj
            return (tuple(cums), ev_acc, int_acc, pr_acc)

        def bin_body(b, carry):
            wait0()
            fire1(b)
            carry = extract(b, carry, g0i_v, g0j_v, 0)
            wait1()
            fire0(b + 1)
            carry = extract(b, carry, g1i_v, g1j_v, NG // 2)
            return carry

        z = jnp.zeros((L,), f32)
        init = (tuple(z for _ in range(NG)), z, z, pr0)
        fire0(jnp.int32(0))
        ev_carry = lax.fori_loop(0, BINS, bin_body, init)
        _, ev_acc, int_acc, pr_acc = ev_carry
        wait0()

        ev_s = jnp.sum(ev_acc)
        int_s = jnp.sum(int_acc)
        pr_s = jnp.sum(pr_acc)
        orow_v[...] = (jnp.where(LANE == 0, ev_s, f32(0.0))
                       + jnp.where(LANE == 1, int_s, f32(0.0))
                       + jnp.where(LANE == 2, pr_s, f32(0.0)))
        pltpu.sync_copy(orow_v, out_hbm.at[pl.ds(wid * L, L)])

    return sc_call


def kernel(event_times, cu_seqlens, node_pairs, x0, v, beta):
    T = event_times.shape[0]
    P = node_pairs.shape[1]
    BINS, NN, D = v.shape
    EV_CHUNK = 1024
    bw = 1.0 / BINS

    cs_pad = jnp.concatenate(
        [cu_seqlens.astype(i32), jnp.full((7,), T, i32)])

    NTILE = (NN + 127) // 128
    vr = (
        jnp.pad(v, ((0, 0), (0, NTILE * 128 - NN), (0, 0)))
        .reshape(BINS, NTILE, 128, D)
        .transpose(0, 1, 3, 2)
        .reshape(BINS * NTILE * D, 128)
    )
    XT = -(-NTILE // 8) * 8
    x0r = (
        jnp.pad(x0[None], ((0, 0), (0, XT * 128 - NN), (0, 0)))
        .reshape(1, XT, 128, D)
        .transpose(0, 1, 3, 2)
        .reshape(XT * D, 128)
    )
    npf = node_pairs.reshape(-1)

    sc_call = _build_sc_call(T, P, BINS, NN, EV_CHUNK)
    parts = sc_call(event_times, cs_pad, npf, x0r, vr).reshape(NW, L)

    ev_delta2 = jnp.sum(parts[:, 0])
    int_raw = jnp.sum(parts[:, 1]) * 0.5
    prior_raw = jnp.sum(parts[:, 2])

    b0 = beta[0]
    integral_term = jnp.exp(b0) * int_raw * bw
    events_term = T * b0 - ev_delta2
    prior_term = 0.5 * PW_ * prior_raw
    return integral_term - events_term + prior_term


# speedup vs baseline: 1.0498x; 1.0498x over previous
import functools
import jax
import jax.numpy as jnp
from jax import lax
from jax.experimental import pallas as pl
from jax.experimental.pallas import tpu as pltpu
from jax.experimental.pallas import tpu_sc as plsc

NC, NS, L = 2, 16, 16
NW = NC * NS
PW_ = 1.0

f32 = jnp.float32
i32 = jnp.int32


def _build_sc_call(T, P, BINS, NN, EV_CHUNK=1024):
    PB = P // NW
    NBK = BINS * PB
    PBF = 2 * PB
    NG = PBF // L
    NTILE = (NN + 127) // 128
    bwf = f32(1.0 / BINS)
    half = f32(0.5 / BINS)
    mesh = plsc.VectorSubcoreMesh(core_axis_name="c", subcore_axis_name="s")

    @functools.partial(
        pl.kernel, mesh=mesh,
        out_type=jax.ShapeDtypeStruct((NW * L,), f32),
        compiler_params=pltpu.CompilerParams(needs_layout_passes=False),
        scratch_types=[
            pltpu.VMEM((136,), i32),
            pltpu.VMEM((PB,), i32),
            pltpu.VMEM((PB,), i32),
            pltpu.VMEM((PB,), i32),
            pltpu.VMEM((PB,), i32),
            pltpu.VMEM((PB,), i32),
            pltpu.VMEM((PB,), i32),
            pltpu.VMEM((PB,), i32),
            pltpu.VMEM((PB,), i32),
            pltpu.VMEM((PB,), i32),
            pltpu.VMEM((PB,), i32),
            pltpu.VMEM((PBF,), i32),
            pltpu.VMEM((PBF,), i32),
            pltpu.VMEM((PB, 128), f32),
            pltpu.VMEM((PB, 128), f32),
            pltpu.VMEM((PB, 128), f32),
            pltpu.VMEM((PB, 128), f32),
            pltpu.VMEM((PBF,), f32),
            pltpu.VMEM((NBK,), f32),
            pltpu.VMEM((NBK,), f32),
            pltpu.VMEM((NBK,), f32),
            pltpu.VMEM((EV_CHUNK,), f32),
            pltpu.VMEM((L,), f32),
            pltpu.VMEM((L,), f32),
            pltpu.SemaphoreType.DMA,
            pltpu.SemaphoreType.DMA,
        ],
    )
    def sc_call(et_hbm, cs_hbm, np_hbm, x0r_hbm, vr_hbm, out_hbm,
                cs_loc, mi_v, mj_v, rtiA_v, rtiB_v, rtjA_v, rtjB_v,
                ixiA_v, ixiB_v, ixjA_v, ixjB_v,
                fci_v, fcj_v, giA_v, giB_v, gjA_v, gjB_v, dx0_v,
                cnt_v, s1_v, s2_v, ev_v, shuf_v, orow_v, semA, semB):
        wid = lax.axis_index("s") * NC + lax.axis_index("c")
        pbase = wid * PB
        LANE = lax.iota(i32, L)
        DUPV = lax.shift_right_logical(LANE, 1)
        SWAPV = lax.bitwise_xor(LANE, 1)
        COLV = lax.bitwise_and(LANE, 1)

        pltpu.sync_copy(cs_hbm.at[pl.ds(pbase, 136)], cs_loc)
        pltpu.sync_copy(np_hbm.at[pl.ds(pbase, PB)], mi_v)
        pltpu.sync_copy(np_hbm.at[pl.ds(P + pbase, PB)], mj_v)

        for g in range(NG):
            sl = pl.ds(g * L, L)
            ni = plsc.load_gather(mi_v, [g * 8 + DUPV])
            nj = plsc.load_gather(mj_v, [g * 8 + DUPV])
            fci_v[sl] = lax.bitwise_and(ni, 127)
            fcj_v[sl] = lax.bitwise_and(nj, 127)
            rti = 2 * lax.shift_right_logical(ni, 7) + COLV
            rtj = 2 * lax.shift_right_logical(nj, 7) + COLV
            slh = pl.ds((g % 8) * L, L)
            if g < 8:
                rtiA_v[slh] = rti
                rtjA_v[slh] = rtj
            else:
                rtiB_v[slh] = rti
                rtjB_v[slh] = rtj

        cp0 = pltpu.async_copy(x0r_hbm.at[rtiA_v], giA_v, semA)
        cp1 = pltpu.async_copy(x0r_hbm.at[rtjA_v], gjA_v, semB)
        cp2 = pltpu.async_copy(x0r_hbm.at[rtiB_v], giB_v, semA)
        cp3 = pltpu.async_copy(x0r_hbm.at[rtjB_v], gjB_v, semB)
        cp0.wait()
        cp1.wait()
        cp2.wait()
        cp3.wait()

        e0 = cs_loc[pl.ds(0, L)][0]
        e1 = cs_loc[pl.ds(120, L)][8]

        pr0 = jnp.zeros((L,), f32)
        for g in range(NG):
            sl = pl.ds(g * L, L)
            rowv = (g % 8) * L + LANE
            if g < 8:
                xi = plsc.load_gather(giA_v, [rowv, fci_v[sl]])
                xj = plsc.load_gather(gjA_v, [rowv, fcj_v[sl]])
            else:
                xi = plsc.load_gather(giB_v, [rowv, fci_v[sl]])
                xj = plsc.load_gather(gjB_v, [rowv, fcj_v[sl]])
            dx0_v[sl] = xi - xj
            pr0 = pr0 + xi * xi + xj * xj

        def zero_body(k, _):
            z = jnp.zeros((L,), f32)
            cnt_v[pl.ds(k * L, L)] = z
            s1_v[pl.ds(k * L, L)] = z
            s2_v[pl.ds(k * L, L)] = z
            return 0
        lax.fori_loop(0, NBK // L, zero_body, 0)

        e0_al = (e0 // 8) * 8
        nch = (e1 - e0_al + (EV_CHUNK - 1)) // EV_CHUNK
        ones = jnp.ones((L,), f32)

        def ev_chunk(ch, _):
            chb = e0_al + ch * EV_CHUNK
            chb_dma = jnp.minimum(chb, T - EV_CHUNK)
            delta = chb - chb_dma
            pltpu.async_copy(
                et_hbm.at[pl.ds(chb_dma, EV_CHUNK)], ev_v, semA).wait()
            for g in range(EV_CHUNK // L):
                gidx = chb + g * L + LANE
                valid = jnp.logical_and(gidx >= e0, gidx < e1)
                t_e = plsc.load_gather(
                    ev_v,
                    [jnp.minimum(delta + g * L + LANE, EV_CHUNK - 1)])
                pos = jnp.zeros((L,), i32)
                for w in (64, 32, 16, 8, 4, 2, 1):
                    cand = pos + w
                    cv = plsc.load_gather(cs_loc, [cand])
                    pos = jnp.where(cv <= gidx, cand, pos)
                b = jnp.clip((t_e / bwf).astype(i32), 0, BINS - 1)
                r = t_e - b.astype(f32) * bwf
                bucket = b * PB + pos
                plsc.addupdate_scatter(cnt_v, [bucket], ones, mask=valid)
                plsc.addupdate_scatter(s1_v, [bucket], r, mask=valid)
                plsc.addupdate_scatter(s2_v, [bucket], r * r, mask=valid)
            return 0
        lax.fori_loop(0, nch, ev_chunk, 0)

        def bin_body(b, carry):
            cums = carry[0]
            ev_acc, int_acc, pr_acc = carry[1], carry[2], carry[3]
            rowoff = 2 * b * NTILE
            for g in range(PB // L):
                sl = pl.ds(g * L, L)
                ixiA_v[sl] = rtiA_v[sl] + rowoff
                ixiB_v[sl] = rtiB_v[sl] + rowoff
                ixjA_v[sl] = rtjA_v[sl] + rowoff
                ixjB_v[sl] = rtjB_v[sl] + rowoff
            cpi = pltpu.async_copy(vr_hbm.at[ixiA_v], giA_v, semA)
            cpj = pltpu.async_copy(vr_hbm.at[ixjA_v], gjA_v, semB)
            cpi2 = pltpu.async_copy(vr_hbm.at[ixiB_v], giB_v, semA)
            cpj2 = pltpu.async_copy(vr_hbm.at[ixjB_v], gjB_v, semB)
            cpi.wait()
            cpj.wait()
            cpi2.wait()
            cpj2.wait()
            new_cums = []
            for g in range(NG):
                sl = pl.ds(g * L, L)
                rowv = (g % 8) * L + LANE
                if g < 8:
                    vi = plsc.load_gather(giA_v, [rowv, fci_v[sl]])
                    vj = plsc.load_gather(gjA_v, [rowv, fcj_v[sl]])
                else:
                    vi = plsc.load_gather(giB_v, [rowv, fci_v[sl]])
                    vj = plsc.load_gather(gjB_v, [rowv, fcj_v[sl]])
                dv = vi - vj
                e_ = dx0_v[sl] + bwf * cums[g]
                new_cums.append(cums[g] + dv)
                pa = e_ * e_
                pb2 = e_ * dv
                pc = dv * dv
                dup = b * PB + g * 8 + DUPV
                c_ = plsc.load_gather(cnt_v, [dup])
                s1 = plsc.load_gather(s1_v, [dup])
                s2 = plsc.load_gather(s2_v, [dup])
                ev_acc = ev_acc + pa * c_ + 2.0 * pb2 * s1 + pc * s2
                d2l = pa + pb2 * bwf + pc * (half * half)
                shuf_v[...] = d2l
                d2s = plsc.load_gather(shuf_v, [SWAPV])
                int_acc = int_acc + jnp.exp(-(d2l + d2s))
                pr_acc = pr_acc + vi * vi + vj * vj
            return (tuple(new_cums), ev_acc, int_acc, pr_acc)

        z = jnp.zeros((L,), f32)
        init = (tuple(z for _ in range(NG)), z, z, pr0)
        _, ev_acc, int_acc, pr_acc = lax.fori_loop(0, BINS, bin_body, init)

        ev_s = jnp.sum(ev_acc)
        int_s = jnp.sum(int_acc)
        pr_s = jnp.sum(pr_acc)
        orow_v[...] = (jnp.where(LANE == 0, ev_s, f32(0.0))
                       + jnp.where(LANE == 1, int_s, f32(0.0))
                       + jnp.where(LANE == 2, pr_s, f32(0.0)))
        pltpu.sync_copy(orow_v, out_hbm.at[pl.ds(wid * L, L)])

    return sc_call


def kernel(event_times, cu_seqlens, node_pairs, x0, v, beta):
    T = event_times.shape[0]
    P = node_pairs.shape[1]
    BINS, NN, D = v.shape
    EV_CHUNK = 1024
    bw = 1.0 / BINS

    cs_pad = jnp.concatenate(
        [cu_seqlens.astype(i32), jnp.full((7,), T, i32)])

    NTILE = (NN + 127) // 128
    vr = (
        jnp.pad(v, ((0, 0), (0, NTILE * 128 - NN), (0, 0)))
        .reshape(BINS, NTILE, 128, D)
        .transpose(0, 1, 3, 2)
        .reshape(BINS * NTILE * D, 128)
    )
    XT = -(-NTILE // 8) * 8
    x0r = (
        jnp.pad(x0[None], ((0, 0), (0, XT * 128 - NN), (0, 0)))
        .reshape(1, XT, 128, D)
        .transpose(0, 1, 3, 2)
        .reshape(XT * D, 128)
    )
    npf = node_pairs.reshape(-1)

    sc_call = _build_sc_call(T, P, BINS, NN, EV_CHUNK)
    parts = sc_call(event_times, cs_pad, npf, x0r, vr).reshape(NW, L)

    ev_delta2 = jnp.sum(parts[:, 0])
    int_raw = jnp.sum(parts[:, 1]) * 0.5
    prior_raw = jnp.sum(parts[:, 2])

    b0 = beta[0]
    integral_term = jnp.exp(b0) * int_raw * bw
    events_term = T * b0 - ev_delta2
    prior_term = 0.5 * PW_ * prior_raw
    return integral_term - events_term + prior_term
